# baseline (speedup 1.0000x reference)
import jax
import jax.numpy as jnp
from jax.experimental import pallas as pl
from jax.experimental.pallas import tpu as pltpu


def _roll_kernel(lab_ref, x_ref, o_ref):
    c = pl.program_id(1)
    h, w = x_ref.shape[2], x_ref.shape[3]
    lab = lab_ref[c]
    x = x_ref[0, 0]

    def roll_h_p():
        o_ref[0, 0] = pltpu.roll(x, 1, 0)

    def roll_h_m():
        o_ref[0, 0] = pltpu.roll(x, h - 1, 0)

    def roll_w_p():
        o_ref[0, 0] = pltpu.roll(x, 1, 1)

    def roll_w_m():
        o_ref[0, 0] = pltpu.roll(x, w - 1, 1)

    jax.lax.switch(lab, [roll_h_p, roll_h_m, roll_w_p, roll_w_m])


@jax.jit
def kernel(x, indices):
    b, c, h, w = x.shape
    q = c // 4
    idx = indices.astype(jnp.int32)
    labels = jnp.zeros((c,), jnp.int32).at[idx].set(jnp.arange(c, dtype=jnp.int32) // q)
    grid_spec = pltpu.PrefetchScalarGridSpec(
        num_scalar_prefetch=1,
        grid=(b, c),
        in_specs=[pl.BlockSpec((1, 1, h, w), lambda i, k, lab: (i, k, 0, 0))],
        out_specs=pl.BlockSpec((1, 1, h, w), lambda i, k, lab: (i, k, 0, 0)),
    )
    return pl.pallas_call(
        _roll_kernel,
        grid_spec=grid_spec,
        out_shape=jax.ShapeDtypeStruct((b, c, h, w), x.dtype),
    )(labels, x)


# speedup vs baseline: 1.7419x; 1.7419x over previous
import functools

import jax
import jax.numpy as jnp
from jax.experimental import pallas as pl
from jax.experimental.pallas import tpu as pltpu


def _roll_kernel(dh_ref, dw_ref, x_ref, o_ref, *, cb):
    c0 = pl.program_id(1) * cb
    for i in range(cb):
        x = x_ref[0, i]
        o_ref[0, i] = pltpu.roll(pltpu.roll(x, dh_ref[c0 + i], 0), dw_ref[c0 + i], 1)


@jax.jit
def kernel(x, indices):
    b, c, h, w = x.shape
    q = c // 4
    cb = 16
    idx = indices.astype(jnp.int32)
    labels = jnp.zeros((c,), jnp.int32).at[idx].set(jnp.arange(c, dtype=jnp.int32) // q)
    dh = jnp.where(labels == 0, 1, jnp.where(labels == 1, h - 1, 0)).astype(jnp.int32)
    dw = jnp.where(labels == 2, 1, jnp.where(labels == 3, w - 1, 0)).astype(jnp.int32)
    grid_spec = pltpu.PrefetchScalarGridSpec(
        num_scalar_prefetch=2,
        grid=(b, c // cb),
        in_specs=[pl.BlockSpec((1, cb, h, w), lambda i, k, dh, dw: (i, k, 0, 0))],
        out_specs=pl.BlockSpec((1, cb, h, w), lambda i, k, dh, dw: (i, k, 0, 0)),
    )
    return pl.pallas_call(
        functools.partial(_roll_kernel, cb=cb),
        grid_spec=grid_spec,
        out_shape=jax.ShapeDtypeStruct((b, c, h, w), x.dtype),
    )(dh, dw, x)


# speedup vs baseline: 1.7903x; 1.0278x over previous
import functools

import jax
import jax.numpy as jnp
from jax.experimental import pallas as pl
from jax.experimental.pallas import tpu as pltpu


def _roll_kernel(lab_ref, x_ref, o_ref, *, cb):
    c0 = pl.program_id(1) * cb
    h, w = x_ref.shape[2], x_ref.shape[3]
    for i in range(cb):
        lab = lab_ref[c0 + i]
        x = x_ref[0, i]

        def roll_h_p(x=x, i=i):
            o_ref[0, i] = pltpu.roll(x, 1, 0)

        def roll_h_m(x=x, i=i):
            o_ref[0, i] = pltpu.roll(x, h - 1, 0)

        def roll_w_p(x=x, i=i):
            o_ref[0, i] = pltpu.roll(x, 1, 1)

        def roll_w_m(x=x, i=i):
            o_ref[0, i] = pltpu.roll(x, w - 1, 1)

        jax.lax.switch(lab, [roll_h_p, roll_h_m, roll_w_p, roll_w_m])


@jax.jit
def kernel(x, indices):
    b, c, h, w = x.shape
    q = c // 4
    cb = 16
    idx = indices.astype(jnp.int32)
    labels = jnp.zeros((c,), jnp.int32).at[idx].set(jnp.arange(c, dtype=jnp.int32) // q)
    grid_spec = pltpu.PrefetchScalarGridSpec(
        num_scalar_prefetch=1,
        grid=(b, c // cb),
        in_specs=[pl.BlockSpec((1, cb, h, w), lambda i, k, lab: (i, k, 0, 0))],
        out_specs=pl.BlockSpec((1, cb, h, w), lambda i, k, lab: (i, k, 0, 0)),
    )
    return pl.pallas_call(
        functools.partial(_roll_kernel, cb=cb),
        grid_spec=grid_spec,
        out_shape=jax.ShapeDtypeStruct((b, c, h, w), x.dtype),
        compiler_params=pltpu.CompilerParams(
            dimension_semantics=("parallel", "parallel"),
        ),
    )(labels, x)


# speedup vs baseline: 1.8506x; 1.0337x over previous
import functools

import jax
import jax.numpy as jnp
from jax.experimental import pallas as pl
from jax.experimental.pallas import tpu as pltpu


def _roll_kernel(lab_ref, x_ref, o_ref, *, cb):
    c0 = pl.program_id(1) * cb
    h, w = x_ref.shape[2], x_ref.shape[3]
    for i in range(cb):
        lab = lab_ref[c0 + i]
        x = x_ref[0, i]

        def roll_h_p(x=x, i=i):
            o_ref[0, i] = pltpu.roll(x, 1, 0)

        def roll_h_m(x=x, i=i):
            o_ref[0, i] = pltpu.roll(x, h - 1, 0)

        def roll_w_p(x=x, i=i):
            o_ref[0, i] = pltpu.roll(x, 1, 1)

        def roll_w_m(x=x, i=i):
            o_ref[0, i] = pltpu.roll(x, w - 1, 1)

        jax.lax.switch(lab, [roll_h_p, roll_h_m, roll_w_p, roll_w_m])


@jax.jit
def kernel(x, indices):
    b, c, h, w = x.shape
    q = c // 4
    cb = 32
    idx = indices.astype(jnp.int32)
    labels = jnp.zeros((c,), jnp.int32).at[idx].set(jnp.arange(c, dtype=jnp.int32) // q)
    grid_spec = pltpu.PrefetchScalarGridSpec(
        num_scalar_prefetch=1,
        grid=(b, c // cb),
        in_specs=[pl.BlockSpec((1, cb, h, w), lambda i, k, lab: (i, k, 0, 0))],
        out_specs=pl.BlockSpec((1, cb, h, w), lambda i, k, lab: (i, k, 0, 0)),
    )
    return pl.pallas_call(
        functools.partial(_roll_kernel, cb=cb),
        grid_spec=grid_spec,
        out_shape=jax.ShapeDtypeStruct((b, c, h, w), x.dtype),
        compiler_params=pltpu.CompilerParams(
            dimension_semantics=("parallel", "parallel"),
        ),
    )(labels, x)


# speedup vs baseline: 1.8901x; 1.0213x over previous
import functools

import jax
import jax.numpy as jnp
from jax.experimental import pallas as pl
from jax.experimental.pallas import tpu as pltpu


def _roll_kernel(lab_ref, x_ref, o_ref, *, cb):
    c0 = pl.program_id(1) * cb
    h, w = x_ref.shape[2], x_ref.shape[3]
    for i in range(cb):
        lab = lab_ref[c0 + i]
        x = x_ref[0, i]

        def roll_h_p(x=x, i=i):
            o_ref[0, i] = pltpu.roll(x, 1, 0)

        def roll_h_m(x=x, i=i):
            o_ref[0, i] = pltpu.roll(x, h - 1, 0)

        def roll_w_p(x=x, i=i):
            o_ref[0, i] = pltpu.roll(x, 1, 1)

        def roll_w_m(x=x, i=i):
            o_ref[0, i] = pltpu.roll(x, w - 1, 1)

        jax.lax.switch(lab, [roll_h_p, roll_h_m, roll_w_p, roll_w_m])


@jax.jit
def kernel(x, indices):
    b, c, h, w = x.shape
    q = c // 4
    cb = 64
    idx = indices.astype(jnp.int32)
    labels = jnp.zeros((c,), jnp.int32).at[idx].set(jnp.arange(c, dtype=jnp.int32) // q)
    grid_spec = pltpu.PrefetchScalarGridSpec(
        num_scalar_prefetch=1,
        grid=(b, c // cb),
        in_specs=[pl.BlockSpec((1, cb, h, w), lambda i, k, lab: (i, k, 0, 0))],
        out_specs=pl.BlockSpec((1, cb, h, w), lambda i, k, lab: (i, k, 0, 0)),
    )
    return pl.pallas_call(
        functools.partial(_roll_kernel, cb=cb),
        grid_spec=grid_spec,
        out_shape=jax.ShapeDtypeStruct((b, c, h, w), x.dtype),
        compiler_params=pltpu.CompilerParams(
            dimension_semantics=("parallel", "parallel"),
        ),
    )(labels, x)


# speedup vs baseline: 1.9272x; 1.0197x over previous
import functools

import jax
import jax.numpy as jnp
from jax.experimental import pallas as pl
from jax.experimental.pallas import tpu as pltpu

L = 4
S = 8
CB = 16


def _roll_kernel(lab_ref, x_ref, o_ref, buf, isems, osems, *, kblocks, total, h, w):
    t = pl.program_id(0)

    def in_copy(tt):
        return pltpu.make_async_copy(
            x_ref.at[tt // kblocks, pl.ds((tt % kblocks) * CB, CB)],
            buf.at[tt % S],
            isems.at[tt % S],
        )

    def out_copy(tt):
        return pltpu.make_async_copy(
            buf.at[tt % S],
            o_ref.at[tt // kblocks, pl.ds((tt % kblocks) * CB, CB)],
            osems.at[tt % S],
        )

    @pl.when(t == 0)
    def _():
        for j in range(L):
            in_copy(j).start()

    @pl.when(t + L < total)
    def _():
        @pl.when(t + L >= S)
        def _():
            out_copy(t + L - S).wait()

        in_copy(t + L).start()

    in_copy(t).wait()

    slot = t % S
    c_base = (t % kblocks) * CB
    for i in range(CB):
        lab = lab_ref[c_base + i]
        x = buf[slot, i]

        def roll_h_p(x=x, i=i):
            buf[slot, i] = pltpu.roll(x, 1, 0)

        def roll_h_m(x=x, i=i):
            buf[slot, i] = pltpu.roll(x, h - 1, 0)

        def roll_w_p(x=x, i=i):
            buf[slot, i] = pltpu.roll(x, 1, 1)

        def roll_w_m(x=x, i=i):
            buf[slot, i] = pltpu.roll(x, w - 1, 1)

        jax.lax.switch(lab, [roll_h_p, roll_h_m, roll_w_p, roll_w_m])

    out_copy(t).start()

    @pl.when(t == total - 1)
    def _():
        for j in range(S):
            out_copy(total - S + j).wait()


@jax.jit
def kernel(x, indices):
    b, c, h, w = x.shape
    q = c // 4
    kblocks = c // CB
    total = b * kblocks
    idx = indices.astype(jnp.int32)
    labels = jnp.zeros((c,), jnp.int32).at[idx].set(jnp.arange(c, dtype=jnp.int32) // q)
    grid_spec = pltpu.PrefetchScalarGridSpec(
        num_scalar_prefetch=1,
        grid=(total,),
        in_specs=[pl.BlockSpec(memory_space=pl.ANY)],
        out_specs=pl.BlockSpec(memory_space=pl.ANY),
        scratch_shapes=[
            pltpu.VMEM((S, CB, h, w), jnp.float32),
            pltpu.SemaphoreType.DMA((S,)),
            pltpu.SemaphoreType.DMA((S,)),
        ],
    )
    return pl.pallas_call(
        functools.partial(_roll_kernel, kblocks=kblocks, total=total, h=h, w=w),
        grid_spec=grid_spec,
        out_shape=jax.ShapeDtypeStruct((b, c, h, w), x.dtype),
    )(labels, x)
